# Initial kernel scaffold; baseline (speedup 1.0000x reference)
import jax
import jax.numpy as jnp
from jax.experimental import pallas as pl
from jax.experimental.pallas import tpu as pltpu

_Z = 0.3


def _matmul_kernel(x_ref, w_ref, out_ref):
    out_ref[...] = jax.lax.dot_general(
        x_ref[...], w_ref[...], (((1,), (0,)), ((), ())),
        precision=jax.lax.Precision.HIGHEST,
        preferred_element_type=jnp.float32)


def _finish_kernel(ls_ref, le_ref, bs_ref, be_ref, labs_ref, labe_ref,
                   am_ref, ps_ref, pe_ref, ms_ref, me_ref, loss_ref):
    n_total = ls_ref.shape[0] * ls_ref.shape[1]
    ls = ls_ref[...] + bs_ref[0, 0]
    le = le_ref[...] + be_ref[0, 0]
    labs = labs_ref[...]
    labe = labe_ref[...]

    weight = (labs >= 0).astype(jnp.float32)

    def bce_sum(lg, tgt):
        per = (jnp.maximum(lg, 0.0) - lg * tgt
               + jnp.log1p(jnp.exp(-jnp.abs(lg))))
        return jnp.sum(weight * per)

    loss = (bce_sum(ls, labs.astype(jnp.float32))
            + bce_sum(le, labe.astype(jnp.float32))) / n_total
    loss_ref[0, 0] = loss

    ps = jax.nn.sigmoid(ls) * weight
    pe = jax.nn.sigmoid(le) * weight
    ps_ref[...] = ps
    pe_ref[...] = pe

    ml = jnp.sum(am_ref[...], axis=1, keepdims=True) - 2
    length = (ml.astype(jnp.float32) * _Z).astype(jnp.int32)
    length = jnp.where(length < 5, 5, length)
    length = jnp.minimum(length, ml * ml)

    def kth_mask(p):
        xi = jax.lax.bitcast_convert_type(p, jnp.int32)
        t = jnp.zeros_like(length)
        for b in range(30, -1, -1):
            cand = t | (1 << b)
            cnt = jnp.sum((xi >= cand).astype(jnp.int32), axis=1,
                          keepdims=True)
            t = jnp.where(cnt >= length, cand, t)
        thr = jax.lax.bitcast_convert_type(t, jnp.float32)
        return (p >= thr).astype(jnp.uint8)

    ms_ref[...] = kth_mask(ps)
    me_ref[...] = kth_mask(pe)


def kernel(table, attention_mask, table_labels_S, table_labels_E,
           W_S, b_S, W_E, b_E):
    B, L, _, D = table.shape
    M = B * L * L
    R = 4096

    x = table.reshape(M, D)
    w = jnp.concatenate([W_S, W_E], axis=1)

    lg = pl.pallas_call(
        _matmul_kernel,
        grid=(M // R,),
        in_specs=[
            pl.BlockSpec((R, D), lambda i: (i, 0)),
            pl.BlockSpec((D, 2), lambda i: (0, 0)),
        ],
        out_specs=pl.BlockSpec((R, 2), lambda i: (i, 0)),
        out_shape=jax.ShapeDtypeStruct((M, 2), jnp.float32),
        compiler_params=pltpu.CompilerParams(
            dimension_semantics=("parallel",)),
    )(x, w)

    lg3 = lg.reshape(B, L * L, 2)
    ls = lg3[..., 0]
    le = lg3[..., 1]
    labs = table_labels_S.reshape(B, L * L)
    labe = table_labels_E.reshape(B, L * L)

    full = lambda s: pl.BlockSpec(s, lambda: (0,) * len(s))
    N = L * L
    ps, pe, ms, me, loss = pl.pallas_call(
        _finish_kernel,
        in_specs=[full((B, N)), full((B, N)), full((1, 1)), full((1, 1)),
                  full((B, N)), full((B, N)), full((B, L))],
        out_specs=[full((B, N)), full((B, N)), full((B, N)), full((B, N)),
                   full((1, 1))],
        out_shape=[
            jax.ShapeDtypeStruct((B, N), jnp.float32),
            jax.ShapeDtypeStruct((B, N), jnp.float32),
            jax.ShapeDtypeStruct((B, N), jnp.uint8),
            jax.ShapeDtypeStruct((B, N), jnp.uint8),
            jax.ShapeDtypeStruct((1, 1), jnp.float32),
        ],
    )(ls, le, b_S.reshape(1, 1), b_E.reshape(1, 1), labs, labe,
      attention_mask)

    logits_S = ls.reshape(B, L, L)
    logits_E = le.reshape(B, L, L)
    S_pred = ps.reshape(B, L, L)
    E_pred = pe.reshape(B, L, L)
    pred_S = (ms != 0).reshape(B, L, L)
    pred_E = (me != 0).reshape(B, L, L)
    return (loss[0, 0], S_pred, E_pred, logits_S, logits_E, pred_S, pred_E)


# speedup vs baseline: 1.1135x; 1.1135x over previous
import jax
import jax.numpy as jnp
from jax.experimental import pallas as pl
from jax.experimental.pallas import tpu as pltpu

_Z = 0.3


def _matmul_kernel(x_ref, w_ref, out_ref):
    out_ref[...] = jax.lax.dot_general(
        x_ref[...], w_ref[...], (((1,), (0,)), ((), ())),
        precision=jax.lax.Precision.HIGHEST,
        preferred_element_type=jnp.float32)


def _finish_kernel(ls_ref, le_ref, bs_ref, be_ref, labs_ref, labe_ref,
                   am_ref, ps_ref, pe_ref, ms_ref, me_ref, loss_ref):
    n_total = ls_ref.shape[0] * ls_ref.shape[1]
    ls = ls_ref[...] + bs_ref[0, 0]
    le = le_ref[...] + be_ref[0, 0]
    labs = labs_ref[...]
    labe = labe_ref[...]

    weight = (labs >= 0).astype(jnp.float32)

    def bce_sum(lg, tgt):
        per = (jnp.maximum(lg, 0.0) - lg * tgt
               + jnp.log1p(jnp.exp(-jnp.abs(lg))))
        return jnp.sum(weight * per)

    loss = (bce_sum(ls, labs.astype(jnp.float32))
            + bce_sum(le, labe.astype(jnp.float32))) / n_total
    loss_ref[...] = loss.reshape(1, 1)

    ps = jax.nn.sigmoid(ls) * weight
    pe = jax.nn.sigmoid(le) * weight
    ps_ref[...] = ps
    pe_ref[...] = pe

    ml = jnp.sum(am_ref[...], axis=1, keepdims=True) - 2
    length = (ml.astype(jnp.float32) * _Z).astype(jnp.int32)
    length = jnp.where(length < 5, 5, length)
    length = jnp.minimum(length, ml * ml)

    def kth_mask(p):
        xi = jax.lax.bitcast_convert_type(p, jnp.int32)
        t = jnp.zeros_like(length)
        for b in range(30, -1, -1):
            cand = t | (1 << b)
            cnt = jnp.sum((xi >= cand).astype(jnp.int32), axis=1,
                          keepdims=True)
            t = jnp.where(cnt >= length, cand, t)
        thr = jax.lax.bitcast_convert_type(t, jnp.float32)
        return (p >= thr).astype(jnp.uint8)

    ms_ref[...] = kth_mask(ps)
    me_ref[...] = kth_mask(pe)


def kernel(table, attention_mask, table_labels_S, table_labels_E,
           W_S, b_S, W_E, b_E):
    B, L, _, D = table.shape
    M = B * L * L
    R = 4096

    x = table.reshape(M, D)
    w = jnp.concatenate([W_S, W_E], axis=1)

    lg = pl.pallas_call(
        _matmul_kernel,
        grid=(M // R,),
        in_specs=[
            pl.BlockSpec((R, D), lambda i: (i, 0)),
            pl.BlockSpec((D, 2), lambda i: (0, 0)),
        ],
        out_specs=pl.BlockSpec((R, 2), lambda i: (i, 0)),
        out_shape=jax.ShapeDtypeStruct((M, 2), jnp.float32),
        compiler_params=pltpu.CompilerParams(
            dimension_semantics=("parallel",)),
    )(x, w)

    lg3 = lg.reshape(B, L * L, 2)
    ls = lg3[..., 0]
    le = lg3[..., 1]
    labs = table_labels_S.reshape(B, L * L)
    labe = table_labels_E.reshape(B, L * L)

    full = lambda s: pl.BlockSpec(s, lambda: (0,) * len(s))
    N = L * L
    ps, pe, ms, me, loss = pl.pallas_call(
        _finish_kernel,
        in_specs=[full((B, N)), full((B, N)), full((1, 1)), full((1, 1)),
                  full((B, N)), full((B, N)), full((B, L))],
        out_specs=[full((B, N)), full((B, N)), full((B, N)), full((B, N)),
                   full((1, 1))],
        out_shape=[
            jax.ShapeDtypeStruct((B, N), jnp.float32),
            jax.ShapeDtypeStruct((B, N), jnp.float32),
            jax.ShapeDtypeStruct((B, N), jnp.uint8),
            jax.ShapeDtypeStruct((B, N), jnp.uint8),
            jax.ShapeDtypeStruct((1, 1), jnp.float32),
        ],
    )(ls, le, b_S.reshape(1, 1), b_E.reshape(1, 1), labs, labe,
      attention_mask)

    logits_S = ls.reshape(B, L, L)
    logits_E = le.reshape(B, L, L)
    S_pred = ps.reshape(B, L, L)
    E_pred = pe.reshape(B, L, L)
    pred_S = (ms != 0).reshape(B, L, L)
    pred_E = (me != 0).reshape(B, L, L)
    return (loss[0, 0], S_pred, E_pred, logits_S, logits_E, pred_S, pred_E)


# speedup vs baseline: 1.5471x; 1.3894x over previous
import jax
import jax.numpy as jnp
from jax.experimental import pallas as pl
from jax.experimental.pallas import tpu as pltpu

_Z = 0.3


def _matmul_kernel(x_ref, w_ref, out_ref):
    x = x_ref[...]
    xhi = x.astype(jnp.bfloat16)
    r1 = x - xhi.astype(jnp.float32)
    xlo = r1.astype(jnp.bfloat16)
    r2 = r1 - xlo.astype(jnp.float32)
    xll = r2.astype(jnp.bfloat16)
    lhs = jnp.concatenate([xhi, xlo, xll], axis=1)
    c = jax.lax.dot_general(
        lhs, w_ref[...], (((1,), (0,)), ((), ())),
        preferred_element_type=jnp.float32)
    out_ref[...] = c[:, 0:2] + c[:, 2:4] + c[:, 4:6]


def _finish_kernel(ls_ref, le_ref, bs_ref, be_ref, labs_ref, labe_ref,
                   am_ref, ps_ref, pe_ref, ms_ref, me_ref, loss_ref):
    n_total = ls_ref.shape[0] * ls_ref.shape[1]
    ls = ls_ref[...] + bs_ref[0, 0]
    le = le_ref[...] + be_ref[0, 0]
    labs = labs_ref[...]
    labe = labe_ref[...]

    weight = (labs >= 0).astype(jnp.float32)

    def bce_sum(lg, tgt):
        per = (jnp.maximum(lg, 0.0) - lg * tgt
               + jnp.log1p(jnp.exp(-jnp.abs(lg))))
        return jnp.sum(weight * per)

    loss = (bce_sum(ls, labs.astype(jnp.float32))
            + bce_sum(le, labe.astype(jnp.float32))) / n_total
    loss_ref[...] = loss.reshape(1, 1)

    ps = jax.nn.sigmoid(ls) * weight
    pe = jax.nn.sigmoid(le) * weight
    ps_ref[...] = ps
    pe_ref[...] = pe

    ml = jnp.sum(am_ref[...], axis=1, keepdims=True) - 2
    length = (ml.astype(jnp.float32) * _Z).astype(jnp.int32)
    length = jnp.where(length < 5, 5, length)
    length = jnp.minimum(length, ml * ml)

    def kth_mask(p):
        xi = jax.lax.bitcast_convert_type(p, jnp.int32)
        t = jnp.zeros_like(length)
        for b in range(30, -1, -1):
            cand = t | (1 << b)
            cnt = jnp.sum((xi >= cand).astype(jnp.int32), axis=1,
                          keepdims=True)
            t = jnp.where(cnt >= length, cand, t)
        thr = jax.lax.bitcast_convert_type(t, jnp.float32)
        return (p >= thr).astype(jnp.uint8)

    ms_ref[...] = kth_mask(ps)
    me_ref[...] = kth_mask(pe)


def kernel(table, attention_mask, table_labels_S, table_labels_E,
           W_S, b_S, W_E, b_E):
    B, L, _, D = table.shape
    M = B * L * L
    R = 4096

    x = table.reshape(M, D)
    w = jnp.concatenate([W_S, W_E], axis=1)
    whi = w.astype(jnp.bfloat16)
    wr1 = w - whi.astype(jnp.float32)
    wlo = wr1.astype(jnp.bfloat16)
    wr2 = wr1 - wlo.astype(jnp.float32)
    wll = wr2.astype(jnp.bfloat16)
    w6 = jnp.concatenate([whi, wlo, wll], axis=1)
    rhs = jnp.concatenate([w6, w6, w6], axis=0)

    lg = pl.pallas_call(
        _matmul_kernel,
        grid=(M // R,),
        in_specs=[
            pl.BlockSpec((R, D), lambda i: (i, 0)),
            pl.BlockSpec((3 * D, 6), lambda i: (0, 0)),
        ],
        out_specs=pl.BlockSpec((R, 2), lambda i: (i, 0)),
        out_shape=jax.ShapeDtypeStruct((M, 2), jnp.float32),
        compiler_params=pltpu.CompilerParams(
            dimension_semantics=("parallel",)),
    )(x, rhs)

    lg3 = lg.reshape(B, L * L, 2)
    ls = lg3[..., 0]
    le = lg3[..., 1]
    labs = table_labels_S.reshape(B, L * L)
    labe = table_labels_E.reshape(B, L * L)

    full = lambda s: pl.BlockSpec(s, lambda: (0,) * len(s))
    N = L * L
    ps, pe, ms, me, loss = pl.pallas_call(
        _finish_kernel,
        in_specs=[full((B, N)), full((B, N)), full((1, 1)), full((1, 1)),
                  full((B, N)), full((B, N)), full((B, L))],
        out_specs=[full((B, N)), full((B, N)), full((B, N)), full((B, N)),
                   full((1, 1))],
        out_shape=[
            jax.ShapeDtypeStruct((B, N), jnp.float32),
            jax.ShapeDtypeStruct((B, N), jnp.float32),
            jax.ShapeDtypeStruct((B, N), jnp.uint8),
            jax.ShapeDtypeStruct((B, N), jnp.uint8),
            jax.ShapeDtypeStruct((1, 1), jnp.float32),
        ],
    )(ls, le, b_S.reshape(1, 1), b_E.reshape(1, 1), labs, labe,
      attention_mask)

    logits_S = ls.reshape(B, L, L)
    logits_E = le.reshape(B, L, L)
    S_pred = ps.reshape(B, L, L)
    E_pred = pe.reshape(B, L, L)
    pred_S = (ms != 0).reshape(B, L, L)
    pred_E = (me != 0).reshape(B, L, L)
    return (loss[0, 0], S_pred, E_pred, logits_S, logits_E, pred_S, pred_E)
